# baseline (speedup 1.0000x reference)
import functools

import jax
import jax.numpy as jnp
from jax import lax
from jax.experimental import pallas as pl
from jax.experimental.pallas import tpu as pltpu
from jax.experimental.pallas import tpu_sc as plsc

NC, NS = 2, 16
NW = NC * NS
D = 32
B = 4096 * 50
BPW = B // NW
CHUNK = 128
NCHUNK = BPW // CHUNK

_mesh = plsc.VectorSubcoreMesh(core_axis_name="c", subcore_axis_name="s")


@functools.partial(
    pl.kernel,
    out_type=jax.ShapeDtypeStruct((B, D), jnp.float32),
    mesh=_mesh,
    scratch_types=[
        pltpu.VMEM((NCHUNK, CHUNK), jnp.int32),
        pltpu.VMEM((CHUNK, D), jnp.float32),
        pltpu.SemaphoreType.DMA,
    ],
    compiler_params=pltpu.CompilerParams(use_tc_tiling_on_sc=False),
)
def _gather_kernel(idx_hbm, table_hbm, out_hbm, idx_v, rows_v, sem):
    wid = lax.axis_index("s") * NC + lax.axis_index("c")
    base = wid * BPW
    pltpu.sync_copy(idx_hbm.at[wid], idx_v)

    def body(j, carry):
        pltpu.async_copy(table_hbm.at[idx_v.at[j]], rows_v, sem).wait()
        pltpu.sync_copy(rows_v, out_hbm.at[pl.ds(base + j * CHUNK, CHUNK)])
        return carry

    lax.fori_loop(0, NCHUNK, body, 0)


def kernel(inputs, table):
    idx = inputs.reshape(NW, NCHUNK, CHUNK).astype(jnp.int32)
    return _gather_kernel(idx, table)


# speedup vs baseline: 1.0574x; 1.0574x over previous
import functools

import jax
import jax.numpy as jnp
from jax import lax
from jax.experimental import pallas as pl
from jax.experimental.pallas import tpu as pltpu
from jax.experimental.pallas import tpu_sc as plsc

NC, NS = 2, 16
NW = NC * NS
D = 32
B = 4096 * 50
BPW = B // NW
CHUNK = 640
NCHUNK = BPW // CHUNK

_mesh = plsc.VectorSubcoreMesh(core_axis_name="c", subcore_axis_name="s")


@functools.partial(
    pl.kernel,
    out_type=jax.ShapeDtypeStruct((B, D), jnp.float32),
    mesh=_mesh,
    scratch_types=[
        pltpu.VMEM((NCHUNK, CHUNK), jnp.int32),
        pltpu.VMEM((2, CHUNK, D), jnp.float32),
        pltpu.SemaphoreType.DMA((2,)),
        pltpu.SemaphoreType.DMA((2,)),
    ],
    compiler_params=pltpu.CompilerParams(use_tc_tiling_on_sc=False),
)
def _gather_kernel(idx_hbm, table_hbm, out_hbm, idx_v, rows_v, gsem, ssem):
    wid = lax.axis_index("s") * NC + lax.axis_index("c")
    base = wid * BPW
    pltpu.sync_copy(idx_hbm.at[wid], idx_v)

    def fire_gather(j):
        pltpu.async_copy(table_hbm.at[idx_v.at[j]], rows_v.at[j % 2],
                         gsem.at[j % 2])

    def fire_store(j):
        pltpu.async_copy(rows_v.at[j % 2],
                         out_hbm.at[pl.ds(base + j * CHUNK, CHUNK)],
                         ssem.at[j % 2])

    def wait_gather(j):
        pltpu.make_async_copy(table_hbm.at[idx_v.at[j]], rows_v.at[j % 2],
                              gsem.at[j % 2]).wait()

    def wait_store(j):
        pltpu.make_async_copy(rows_v.at[j % 2],
                              out_hbm.at[pl.ds(base + j * CHUNK, CHUNK)],
                              ssem.at[j % 2]).wait()

    fire_gather(0)
    for j in range(NCHUNK):
        if j + 1 < NCHUNK:
            if j >= 1:
                wait_store(j - 1)
            fire_gather(j + 1)
        wait_gather(j)
        fire_store(j)
    wait_store(NCHUNK - 2)
    wait_store(NCHUNK - 1)


def kernel(inputs, table):
    idx = inputs.reshape(NW, NCHUNK, CHUNK).astype(jnp.int32)
    return _gather_kernel(idx, table)
